# Initial kernel scaffold; baseline (speedup 1.0000x reference)
import jax
import jax.numpy as jnp
from jax.experimental import pallas as pl


def kernel(hidden_states, W, scale):
    raise NotImplementedError("write your pallas kernel here")


# speedup vs baseline: 2.5275x; 2.5275x over previous
import functools

import jax
import jax.numpy as jnp
from jax.experimental import pallas as pl

HIDDEN = 2816
NUM_EXPERTS = 128
TOP_K = 8
EPS = 1e-6
BLK = 512


def _router_body(x_ref, w_ref, scale_ref, ow_ref, oi_ref):
    x = x_ref[...]
    ssq = jnp.sum(x * x, axis=1, keepdims=True)
    r = jax.lax.rsqrt(ssq * (1.0 / HIDDEN) + EPS) * (HIDDEN ** -0.5)
    normed = (x * r) * scale_ref[...]
    scores = jax.lax.dot_general(
        normed, w_ref[...],
        dimension_numbers=(((1,), (1,)), ((), ())),
        preferred_element_type=jnp.float32,
    )

    lane = jax.lax.broadcasted_iota(jnp.int32, scores.shape, 1)
    neg_inf = jnp.float32(-jnp.inf)
    vals = scores
    tops = []
    idxs = []
    for _ in range(TOP_K):
        m = jnp.max(vals, axis=1, keepdims=True)
        hit = vals == m
        idx = jnp.min(jnp.where(hit, lane, NUM_EXPERTS), axis=1, keepdims=True)
        tops.append(m)
        idxs.append(idx)
        vals = jnp.where(lane == idx, neg_inf, vals)
    top = jnp.concatenate(tops, axis=1)
    e = jnp.exp(top - tops[0])
    ow_ref[...] = e / jnp.sum(e, axis=1, keepdims=True)
    oi_ref[...] = jnp.concatenate(idxs, axis=1)


@jax.jit
def kernel(hidden_states, W, scale):
    b, s, h = hidden_states.shape
    n_tok = b * s
    x = hidden_states.reshape(n_tok, h)
    grid = (n_tok // BLK,)
    ow, oi = pl.pallas_call(
        _router_body,
        grid=grid,
        in_specs=[
            pl.BlockSpec((BLK, h), lambda i: (i, 0)),
            pl.BlockSpec((NUM_EXPERTS, h), lambda i: (0, 0)),
            pl.BlockSpec((1, h), lambda i: (0, 0)),
        ],
        out_specs=[
            pl.BlockSpec((BLK, TOP_K), lambda i: (i, 0)),
            pl.BlockSpec((BLK, TOP_K), lambda i: (i, 0)),
        ],
        out_shape=[
            jax.ShapeDtypeStruct((n_tok, TOP_K), jnp.float32),
            jax.ShapeDtypeStruct((n_tok, TOP_K), jnp.int32),
        ],
    )(x, W, scale.reshape(1, h))
    return ow.reshape(b, s, TOP_K), oi.reshape(b, s, TOP_K)
